# baseline (speedup 1.0000x reference)
import functools

import jax
import jax.numpy as jnp
from jax import lax
from jax.experimental import pallas as pl
from jax.experimental.pallas import tpu as pltpu
from jax.experimental.pallas import tpu_sc as plsc

B = 32
SH, SW = 32, 32
L = 128
D = 1024
T = SH * SW + L
NPG = 16
NBG = 2
PPW = T // NPG
BPW = B // NBG
CR = 24
NCI = PPW // CR
NT = BPW * NCI
NCOL = D // 16


def _fuse(xidx, aux, aidx, bidx, embedding):
    mesh = plsc.VectorSubcoreMesh(core_axis_name="c", subcore_axis_name="s")

    @functools.partial(
        pl.kernel,
        mesh=mesh,
        out_type=jax.ShapeDtypeStruct((B, T, D), jnp.float32),
        scratch_types=[
            pltpu.VMEM((NT, CR), jnp.int32),
            pltpu.VMEM((NCI, CR), jnp.int32),
            pltpu.VMEM((NCI, CR), jnp.int32),
            pltpu.VMEM((PPW, D), jnp.float32),
            pltpu.VMEM((CR, D), jnp.float32),
            pltpu.VMEM((CR, D), jnp.float32),
            pltpu.SemaphoreType.DMA,
            pltpu.SemaphoreType.DMA,
            pltpu.SemaphoreType.DMA,
            pltpu.SemaphoreType.DMA,
            pltpu.SemaphoreType.DMA,
        ],
    )
    def k(x_hbm, aux_hbm, ai_hbm, bi_hbm, emb_hbm, out_hbm,
          idx_v, ai_v, bi_v, pos_v, buf_a, buf_b,
          sem0, sg_a, sg_b, sw_a, sw_b):
        pg = lax.axis_index("s")
        bg = lax.axis_index("c")
        p0 = pl.multiple_of(pg * PPW, 8)

        pltpu.sync_copy(x_hbm.at[pg, bg], idx_v)
        pltpu.sync_copy(ai_hbm.at[pg], ai_v)
        pltpu.sync_copy(bi_hbm.at[pg], bi_v)

        for kk in range(NCI):
            pltpu.async_copy(
                aux_hbm.at[ai_v.at[kk]],
                pos_v.at[pl.ds(kk * CR, CR)], sem0).wait()
            pltpu.async_copy(
                aux_hbm.at[bi_v.at[kk]],
                buf_b, sem0).wait()

            def posadd(j, c, kk=kk):
                for cc in range(NCOL):
                    sl = pl.ds(cc * 16, 16)
                    pos_v[kk * CR + j, sl] = pos_v[kk * CR + j, sl] + buf_b[j, sl]
                return c

            lax.fori_loop(0, CR, posadd, 0)

        def issue_gather(t, buf, sg):
            pltpu.async_copy(emb_hbm.at[idx_v.at[t]], buf, sg)

        def wait_gather(buf, sg):
            pltpu.make_async_copy(
                emb_hbm.at[idx_v.at[0]], buf, sg).wait()

        def issue_write(t, buf, sw):
            bl = t // NCI
            ci = t % NCI
            bglob = bg * BPW + bl
            off = pl.multiple_of(p0 + ci * CR, 8)
            pltpu.async_copy(buf, out_hbm.at[bglob, pl.ds(off, CR)], sw)

        def wait_write(buf, sw):
            pltpu.make_async_copy(buf, out_hbm.at[0, pl.ds(0, CR)], sw).wait()

        def slot(t, buf, sg, sw, obuf, osg, osw):
            @pl.when(t + 1 < NT)
            def _():
                @pl.when(t >= 1)
                def _():
                    wait_write(obuf, osw)
                issue_gather(t + 1, obuf, osg)

            wait_gather(buf, sg)
            ci = t % NCI

            def addb(j, c):
                row = ci * CR + j
                for cc in range(NCOL):
                    sl = pl.ds(cc * 16, 16)
                    buf[j, sl] = buf[j, sl] + pos_v[row, sl]
                return c

            lax.fori_loop(0, CR, addb, 0)
            issue_write(t, buf, sw)

        issue_gather(0, buf_a, sg_a)

        def ibody(i, c):
            t0 = 2 * i
            slot(t0, buf_a, sg_a, sw_a, buf_b, sg_b, sw_b)
            slot(t0 + 1, buf_b, sg_b, sw_b, buf_a, sg_a, sw_a)
            return c

        lax.fori_loop(0, NT // 2, ibody, 0)
        wait_write(buf_a, sw_a)
        wait_write(buf_b, sw_b)

    return k(xidx, aux, aidx, bidx, embedding)


@jax.jit
def kernel(x, embedding, height_emb, width_emb, length_emb):
    p = jnp.arange(T, dtype=jnp.int32)
    img = p < SH * SW
    aidx = jnp.where(img, p // SW, 2 * SH + (p - SH * SW)).astype(jnp.int32)
    bidx = jnp.where(img, SH + (p % SW), 2 * SH + L).astype(jnp.int32)
    aux = jnp.concatenate(
        [height_emb, width_emb, length_emb,
         jnp.zeros((1, D), jnp.float32)], axis=0)
    xidx = (x.astype(jnp.int32)
            .reshape(NBG, BPW, NPG, NCI, CR).transpose(2, 0, 1, 3, 4)
            .reshape(NPG, NBG, NT, CR))
    return _fuse(xidx, aux,
                 aidx.reshape(NPG, NCI, CR), bidx.reshape(NPG, NCI, CR),
                 embedding)


# speedup vs baseline: 1.1146x; 1.1146x over previous
import functools

import jax
import jax.numpy as jnp
from jax import lax
from jax.experimental import pallas as pl
from jax.experimental.pallas import tpu as pltpu
from jax.experimental.pallas import tpu_sc as plsc

B = 32
SH, SW = 32, 32
L = 128
D = 1024
T = SH * SW + L
NPG = 16
NBG = 2
PPW = T // NPG
BPW = B // NBG
CR = 8
NCI = PPW // CR
NT = BPW * NCI
NR = 4
NCOL = D // 16


def _fuse(xidx, aux, aidx, bidx, embedding):
    mesh = plsc.VectorSubcoreMesh(core_axis_name="c", subcore_axis_name="s")

    @functools.partial(
        pl.kernel,
        mesh=mesh,
        out_type=jax.ShapeDtypeStruct((B, T, D), jnp.float32),
        scratch_types=[
            pltpu.VMEM((NT, CR), jnp.int32),
            pltpu.VMEM((NCI, CR), jnp.int32),
            pltpu.VMEM((NCI, CR), jnp.int32),
            pltpu.VMEM((PPW, D), jnp.float32),
            pltpu.VMEM((CR, D), jnp.float32),
            pltpu.VMEM((CR, D), jnp.float32),
            pltpu.VMEM((CR, D), jnp.float32),
            pltpu.VMEM((CR, D), jnp.float32),
            pltpu.SemaphoreType.DMA,
            pltpu.SemaphoreType.DMA,
            pltpu.SemaphoreType.DMA,
            pltpu.SemaphoreType.DMA,
            pltpu.SemaphoreType.DMA,
            pltpu.SemaphoreType.DMA,
            pltpu.SemaphoreType.DMA,
            pltpu.SemaphoreType.DMA,
            pltpu.SemaphoreType.DMA,
        ],
    )
    def k(x_hbm, aux_hbm, ai_hbm, bi_hbm, emb_hbm, out_hbm,
          idx_v, ai_v, bi_v, pos_v, buf0, buf1, buf2, buf3,
          sem0, sg0, sg1, sg2, sg3, sw0, sw1, sw2, sw3):
        pg = lax.axis_index("s")
        bg = lax.axis_index("c")
        p0 = pl.multiple_of(pg * PPW, 8)

        bufs = (buf0, buf1, buf2, buf3)
        sgat = (sg0, sg1, sg2, sg3)
        swri = (sw0, sw1, sw2, sw3)

        pltpu.sync_copy(x_hbm.at[pg, bg], idx_v)
        pltpu.sync_copy(ai_hbm.at[pg], ai_v)
        pltpu.sync_copy(bi_hbm.at[pg], bi_v)

        for kk in range(NCI):
            pltpu.async_copy(
                aux_hbm.at[ai_v.at[kk]],
                pos_v.at[pl.ds(kk * CR, CR)], sem0).wait()
            pltpu.async_copy(aux_hbm.at[bi_v.at[kk]], buf0, sem0).wait()

            def posadd(j, c, kk=kk):
                for cc in range(NCOL):
                    sl = pl.ds(cc * 16, 16)
                    plsc.addupdate(pos_v.at[kk * CR + j, sl], buf0[j, sl])
                return c

            lax.fori_loop(0, CR, posadd, 0)

        def issue_gather(t, r):
            pltpu.async_copy(emb_hbm.at[idx_v.at[t]], bufs[r], sgat[r])

        def wait_gather(r):
            pltpu.make_async_copy(emb_hbm.at[idx_v.at[0]],
                                  bufs[r], sgat[r]).wait()

        def issue_write(t, r):
            bl = t // NCI
            ci = t % NCI
            bglob = bg * BPW + bl
            off = pl.multiple_of(p0 + ci * CR, 8)
            pltpu.async_copy(bufs[r], out_hbm.at[bglob, pl.ds(off, CR)],
                             swri[r])

        def wait_write(r):
            pltpu.make_async_copy(bufs[r], out_hbm.at[0, pl.ds(0, CR)],
                                  swri[r]).wait()

        def slot(t, r):
            @pl.when(t + (NR - 1) < NT)
            def _():
                @pl.when(t >= 1)
                def _():
                    wait_write((r + NR - 1) % NR)
                issue_gather(t + (NR - 1), (r + NR - 1) % NR)

            wait_gather(r)
            ci = t % NCI

            def addb(j, c):
                row = ci * CR + j
                for cc in range(NCOL):
                    sl = pl.ds(cc * 16, 16)
                    plsc.addupdate(bufs[r].at[j, sl], pos_v[row, sl])
                return c

            lax.fori_loop(0, CR, addb, 0)
            issue_write(t, r)

        for t in range(NR - 1):
            issue_gather(t, t)

        def ibody(i, c):
            for r in range(NR):
                slot(NR * i + r, r)
            return c

        lax.fori_loop(0, NT // NR, ibody, 0)
        for r in range(NR):
            wait_write(r)

    return k(xidx, aux, aidx, bidx, embedding)


@jax.jit
def kernel(x, embedding, height_emb, width_emb, length_emb):
    p = jnp.arange(T, dtype=jnp.int32)
    img = p < SH * SW
    aidx = jnp.where(img, p // SW, 2 * SH + (p - SH * SW)).astype(jnp.int32)
    bidx = jnp.where(img, SH + (p % SW), 2 * SH + L).astype(jnp.int32)
    aux = jnp.concatenate(
        [height_emb, width_emb, length_emb,
         jnp.zeros((1, D), jnp.float32)], axis=0)
    xidx = (x.astype(jnp.int32)
            .reshape(NBG, BPW, NPG, NCI, CR).transpose(2, 0, 1, 3, 4)
            .reshape(NPG, NBG, NT, CR))
    return _fuse(xidx, aux,
                 aidx.reshape(NPG, NCI, CR), bidx.reshape(NPG, NCI, CR),
                 embedding)


# speedup vs baseline: 1.3483x; 1.2097x over previous
import functools

import jax
import jax.numpy as jnp
from jax import lax
from jax.experimental import pallas as pl
from jax.experimental.pallas import tpu as pltpu
from jax.experimental.pallas import tpu_sc as plsc

B = 32
SH, SW = 32, 32
L = 128
D = 1024
T = SH * SW + L
NK = 4
BK = B // NK
NPG = 16
NBG = 2
PPW = T // NPG
BPW = BK // NBG
CR = 8
NCI = PPW // CR
NT = BPW * NCI
NR = 4


def _sc_gather(xidx, embedding):
    mesh = plsc.VectorSubcoreMesh(core_axis_name="c", subcore_axis_name="s")

    @functools.partial(
        pl.kernel,
        mesh=mesh,
        out_type=jax.ShapeDtypeStruct((BK, T, D), jnp.float32),
        scratch_types=[
            pltpu.VMEM((NT, CR), jnp.int32),
            pltpu.VMEM((CR, D), jnp.float32),
            pltpu.VMEM((CR, D), jnp.float32),
            pltpu.VMEM((CR, D), jnp.float32),
            pltpu.VMEM((CR, D), jnp.float32),
            pltpu.SemaphoreType.DMA,
            pltpu.SemaphoreType.DMA,
            pltpu.SemaphoreType.DMA,
            pltpu.SemaphoreType.DMA,
            pltpu.SemaphoreType.DMA,
            pltpu.SemaphoreType.DMA,
            pltpu.SemaphoreType.DMA,
            pltpu.SemaphoreType.DMA,
        ],
    )
    def k(x_hbm, emb_hbm, out_hbm,
          idx_v, buf0, buf1, buf2, buf3,
          sg0, sg1, sg2, sg3, sw0, sw1, sw2, sw3):
        pg = lax.axis_index("s")
        bg = lax.axis_index("c")
        p0 = pl.multiple_of(pg * PPW, 8)

        bufs = (buf0, buf1, buf2, buf3)
        sgat = (sg0, sg1, sg2, sg3)
        swri = (sw0, sw1, sw2, sw3)

        pltpu.sync_copy(x_hbm.at[pg, bg], idx_v)

        def issue_gather(t, r):
            pltpu.async_copy(emb_hbm.at[idx_v.at[t]], bufs[r], sgat[r])

        def wait_gather(r):
            pltpu.make_async_copy(emb_hbm.at[idx_v.at[0]],
                                  bufs[r], sgat[r]).wait()

        def issue_write(t, r):
            bl = t // NCI
            ci = t % NCI
            bglob = bg * BPW + bl
            off = pl.multiple_of(p0 + ci * CR, 8)
            pltpu.async_copy(bufs[r], out_hbm.at[bglob, pl.ds(off, CR)],
                             swri[r])

        def wait_write(r):
            pltpu.make_async_copy(bufs[r], out_hbm.at[0, pl.ds(0, CR)],
                                  swri[r]).wait()

        def slot(t, r):
            @pl.when(t + (NR - 1) < NT)
            def _():
                @pl.when(t >= 1)
                def _():
                    wait_write((r + NR - 1) % NR)
                issue_gather(t + (NR - 1), (r + NR - 1) % NR)

            wait_gather(r)
            issue_write(t, r)

        for t in range(NR - 1):
            issue_gather(t, t)

        def ibody(i, c):
            for r in range(NR):
                slot(NR * i + r, r)
            return c

        lax.fori_loop(0, NT // NR, ibody, 0)
        for r in range(NR):
            wait_write(r)

    return k(xidx, embedding)


def _pos_body(h_ref, w_ref, l_ref, out_ref):
    hh = jnp.broadcast_to(h_ref[...][:, None, :], (SH, SW, D))
    ww = jnp.broadcast_to(w_ref[...][None, :, :], (SH, SW, D))
    out_ref[: SH * SW, :] = (hh + ww).reshape(SH * SW, D)
    out_ref[SH * SW :, :] = l_ref[...]


def _tc_pos(height_emb, width_emb, length_emb):
    return pl.pallas_call(
        _pos_body,
        out_shape=jax.ShapeDtypeStruct((T, D), jnp.float32),
    )(height_emb, width_emb, length_emb)


def _add_body(g_ref, p_ref, out_ref):
    out_ref[...] = g_ref[...] + p_ref[...][None]


def _tc_add(g, pos):
    return pl.pallas_call(
        _add_body,
        grid=(BK,),
        in_specs=[
            pl.BlockSpec((1, T, D), lambda b: (b, 0, 0)),
            pl.BlockSpec((T, D), lambda b: (0, 0)),
        ],
        out_specs=pl.BlockSpec((1, T, D), lambda b: (b, 0, 0)),
        out_shape=jax.ShapeDtypeStruct((BK, T, D), jnp.float32),
    )(g, pos)


@jax.jit
def kernel(x, embedding, height_emb, width_emb, length_emb):
    xidx = (x.astype(jnp.int32)
            .reshape(NK, NBG, BPW, NPG, NCI, CR)
            .transpose(0, 3, 1, 2, 4, 5)
            .reshape(NK, NPG, NBG, NT, CR))
    pos = _tc_pos(height_emb, width_emb, length_emb)
    outs = []
    for kc in range(NK):
        g = _sc_gather(xidx[kc], embedding)
        outs.append(_tc_add(g, pos))
    return jnp.concatenate(outs, axis=0)


# speedup vs baseline: 1.9549x; 1.4499x over previous
import functools

import jax
import jax.numpy as jnp
from jax import lax
from jax.experimental import pallas as pl
from jax.experimental.pallas import tpu as pltpu
from jax.experimental.pallas import tpu_sc as plsc

B = 32
SH, SW = 32, 32
L = 128
D = 1024
T = SH * SW + L
NK = 4
BK = B // NK
NPG = 16
NBG = 2
PPW = T // NPG
BPW = BK // NBG
CR = 24
NCI = PPW // CR
NT = BPW * NCI
NR = 4


def _sc_gather(xidx, embedding):
    mesh = plsc.VectorSubcoreMesh(core_axis_name="c", subcore_axis_name="s")

    @functools.partial(
        pl.kernel,
        mesh=mesh,
        out_type=jax.ShapeDtypeStruct((BK, T, D), jnp.float32),
        scratch_types=[
            pltpu.VMEM((NT, CR), jnp.int32),
            pltpu.VMEM((CR, D), jnp.float32),
            pltpu.VMEM((CR, D), jnp.float32),
            pltpu.VMEM((CR, D), jnp.float32),
            pltpu.VMEM((CR, D), jnp.float32),
            pltpu.SemaphoreType.DMA,
            pltpu.SemaphoreType.DMA,
            pltpu.SemaphoreType.DMA,
            pltpu.SemaphoreType.DMA,
            pltpu.SemaphoreType.DMA,
            pltpu.SemaphoreType.DMA,
            pltpu.SemaphoreType.DMA,
            pltpu.SemaphoreType.DMA,
        ],
    )
    def k(x_hbm, emb_hbm, out_hbm,
          idx_v, buf0, buf1, buf2, buf3,
          sg0, sg1, sg2, sg3, sw0, sw1, sw2, sw3):
        pg = lax.axis_index("s")
        bg = lax.axis_index("c")
        p0 = pl.multiple_of(pg * PPW, 8)

        bufs = (buf0, buf1, buf2, buf3)
        sgat = (sg0, sg1, sg2, sg3)
        swri = (sw0, sw1, sw2, sw3)

        pltpu.sync_copy(x_hbm.at[pg, bg], idx_v)

        def issue_gather(t, r):
            pltpu.async_copy(emb_hbm.at[idx_v.at[t]], bufs[r], sgat[r])

        def wait_gather(r):
            pltpu.make_async_copy(emb_hbm.at[idx_v.at[0]],
                                  bufs[r], sgat[r]).wait()

        def issue_write(t, r):
            bl = t // NCI
            ci = t % NCI
            bglob = bg * BPW + bl
            off = pl.multiple_of(p0 + ci * CR, 8)
            pltpu.async_copy(bufs[r], out_hbm.at[bglob, pl.ds(off, CR)],
                             swri[r])

        def wait_write(r):
            pltpu.make_async_copy(bufs[r], out_hbm.at[0, pl.ds(0, CR)],
                                  swri[r]).wait()

        def slot(t, r):
            @pl.when(t + (NR - 1) < NT)
            def _():
                @pl.when(t >= 1)
                def _():
                    wait_write((r + NR - 1) % NR)
                issue_gather(t + (NR - 1), (r + NR - 1) % NR)

            wait_gather(r)
            issue_write(t, r)

        for t in range(NR - 1):
            issue_gather(t, t)

        def ibody(i, c):
            for r in range(NR):
                slot(NR * i + r, r)
            return c

        lax.fori_loop(0, NT // NR, ibody, 0)
        for r in range(NR):
            wait_write(r)

    return k(xidx, embedding)


def _pos_body(h_ref, w_ref, l_ref, out_ref):
    hh = jnp.broadcast_to(h_ref[...][:, None, :], (SH, SW, D))
    ww = jnp.broadcast_to(w_ref[...][None, :, :], (SH, SW, D))
    out_ref[: SH * SW, :] = (hh + ww).reshape(SH * SW, D)
    out_ref[SH * SW :, :] = l_ref[...]


def _tc_pos(height_emb, width_emb, length_emb):
    return pl.pallas_call(
        _pos_body,
        out_shape=jax.ShapeDtypeStruct((T, D), jnp.float32),
    )(height_emb, width_emb, length_emb)


def _add_first_body(g_ref, p_ref, out_ref):
    out_ref[...] = g_ref[...] + p_ref[...][None]


def _add_next_body(prev_ref, g_ref, p_ref, out_ref):
    del prev_ref
    out_ref[...] = g_ref[...] + p_ref[...][None]


def _tc_add(out_prev, g, pos, kc):
    if kc == 0:
        return pl.pallas_call(
            _add_first_body,
            grid=(BK,),
            in_specs=[
                pl.BlockSpec((1, T, D), lambda b: (b, 0, 0)),
                pl.BlockSpec((T, D), lambda b: (0, 0)),
            ],
            out_specs=pl.BlockSpec((1, T, D), lambda b: (b, 0, 0)),
            out_shape=jax.ShapeDtypeStruct((B, T, D), jnp.float32),
        )(g, pos)
    return pl.pallas_call(
        _add_next_body,
        grid=(BK,),
        in_specs=[
            pl.BlockSpec(memory_space=pl.ANY),
            pl.BlockSpec((1, T, D), lambda b: (b, 0, 0)),
            pl.BlockSpec((T, D), lambda b: (0, 0)),
        ],
        out_specs=pl.BlockSpec((1, T, D),
                               lambda b, kc=kc: (kc * BK + b, 0, 0)),
        out_shape=jax.ShapeDtypeStruct((B, T, D), jnp.float32),
        input_output_aliases={0: 0},
    )(out_prev, g, pos)


@jax.jit
def kernel(x, embedding, height_emb, width_emb, length_emb):
    xidx = (x.astype(jnp.int32)
            .reshape(NK, NBG, BPW, NPG, NCI, CR)
            .transpose(0, 3, 1, 2, 4, 5)
            .reshape(NK, NPG, NBG, NT, CR))
    pos = _tc_pos(height_emb, width_emb, length_emb)
    gs = [_sc_gather(xidx[kc], embedding) for kc in range(NK)]
    out = None
    for kc in range(NK):
        out = _tc_add(out, gs[kc], pos, kc)
    return out
